# baseline (speedup 1.0000x reference)
import functools

import jax
import jax.numpy as jnp
from jax import lax
from jax.experimental import pallas as pl
from jax.experimental.pallas import tpu as pltpu
from jax.experimental.pallas import tpu_sc as plsc

H = 128
_NNODES = {"drug": 10000, "protein": 10000, "disease": 2048}
_RELS = [
    ("e_d_t_dr", "disease", "drug"),
    ("e_d_m_dr", "disease", "drug"),
    ("e_d_p", "disease", "protein"),
    ("e_dr_t_d", "drug", "disease"),
    ("e_dr_m_d", "drug", "disease"),
    ("e_p_d", "protein", "disease"),
    ("e_DDI", "drug", "drug"),
    ("e_PPI", "protein", "protein"),
]

_NW = 32
_C = 4096
_G = 32
_LANES = 16


def _rup(x, m):
    return (x + m - 1) // m * m


_RPW = {nt: _rup(_NNODES[nt], _NW) // _NW for nt in _NNODES}
_RMAX = max(_RPW.values())


def _seg_body(*refs):
    m_refs = refs[0:8]
    src_refs = refs[8:16]
    dst_refs = refs[16:24]
    out_refs = refs[24:32]
    srcbuf, dstbuf, sel_src, sel_dst, rows, acc, sem = refs[32:]

    wid = lax.axis_index("s") * 2 + lax.axis_index("c")
    iota = lax.iota(jnp.int32, _LANES)
    zeros16 = jnp.zeros((_LANES,), jnp.float32)
    sent_src = wid * 8

    for r in range(8):
        _, _, dt = _RELS[r]
        R = _RPW[dt]
        lo = wid * R
        ne_pad = src_refs[r].shape[0]
        n_chunks = ne_pad // _C

        def zero_body(j, _, acc=acc):
            acc[pl.ds(j * _LANES, _LANES)] = zeros16
            return 0
        lax.fori_loop(0, (R + 1) * (H // _LANES), zero_body, 0)

        def chunk_body(c, _, r=r, R=R, lo=lo):
            off = c * _C
            pltpu.sync_copy(src_refs[r].at[pl.ds(off, _C)], srcbuf)
            pltpu.sync_copy(dst_refs[r].at[pl.ds(off, _C)], dstbuf)

            def scan_body(i, cnt_vec):
                d = dstbuf[pl.ds(i * _LANES, _LANES)]
                s = srcbuf[pl.ds(i * _LANES, _LANES)]
                rel = d - lo
                mask = plsc.bitcast(rel, jnp.uint32) < jnp.uint32(R)
                key = jnp.where(mask, 0, 1)
                packed = (s << 9) | jnp.where(mask, rel, R)
                _, sorted_packed = plsc.sort_key_val(key, packed)
                posn = cnt_vec + iota
                plsc.store_scatter(sel_src, [posn], sorted_packed >> 9)
                plsc.store_scatter(sel_dst, [posn], sorted_packed & 511)
                return cnt_vec + plsc.all_reduce_population_count(mask)

            cnt_vec = lax.fori_loop(0, _C // _LANES, scan_body,
                                    jnp.zeros((_LANES,), jnp.int32))

            pad_rel = jnp.full((_LANES,), R, jnp.int32)
            pad_src = jnp.full((_LANES,), sent_src, jnp.int32)
            plsc.store_scatter(sel_dst, [cnt_vec + iota], pad_rel)
            plsc.store_scatter(sel_src, [cnt_vec + iota], pad_src)
            plsc.store_scatter(sel_dst, [cnt_vec + 16 + iota], pad_rel)
            plsc.store_scatter(sel_src, [cnt_vec + 16 + iota], pad_src)
            cnt = jnp.max(cnt_vec)
            n_grp = (cnt + _G - 1) // _G

            def merge_grp(grp, _, r=r):
                idx = sel_src.at[pl.ds(grp * _G, _G)]
                pltpu.async_copy(m_refs[r].at[idx], rows, sem).wait()

                def edge_body(e, _):
                    e_vec = jnp.full((_LANES,), e, jnp.int32)
                    de = plsc.load_gather(
                        sel_dst, [jnp.full((_LANES,), grp * _G + e, jnp.int32)])
                    base = de * H
                    for k in range(H // _LANES):
                        ck = iota + (k * _LANES)
                        rv = plsc.load_gather(rows, [e_vec, ck])
                        aidx = base + ck
                        av = plsc.load_gather(acc, [aidx])
                        plsc.store_scatter(acc, [aidx], jnp.maximum(av, rv))
                    return 0

                lax.fori_loop(0, _G, edge_body, 0)
                return 0

            lax.fori_loop(0, n_grp, merge_grp, 0)
            return 0

        lax.fori_loop(0, n_chunks, chunk_body, 0)

        pltpu.sync_copy(acc.at[pl.ds(0, R * H)],
                        out_refs[r].at[pl.ds(wid * R * H, R * H)])


def _segment_max_layer(ms, srcs, dsts):
    out_types = [
        jax.ShapeDtypeStruct((_NW * _RPW[_RELS[r][2]] * H,), jnp.float32)
        for r in range(8)
    ]
    mesh = plsc.VectorSubcoreMesh(core_axis_name="c", subcore_axis_name="s")
    fn = pl.kernel(
        _seg_body,
        out_type=out_types,
        mesh=mesh,
        scratch_types=[
            pltpu.VMEM((_C,), jnp.int32),
            pltpu.VMEM((_C,), jnp.int32),
            pltpu.VMEM((_C + 32,), jnp.int32),
            pltpu.VMEM((_C + 32,), jnp.int32),
            pltpu.VMEM((_G, H), jnp.float32),
            pltpu.VMEM(((_RMAX + 1) * H,), jnp.float32),
            pltpu.SemaphoreType.DMA,
        ],
        compiler_params=pltpu.CompilerParams(needs_layout_passes=False),
    )
    outs = fn(*ms, *srcs, *dsts)
    res = []
    for r, o in enumerate(outs):
        n_dst = _NNODES[_RELS[r][2]]
        res.append(o.reshape(_NW * _RPW[_RELS[r][2]], H)[:n_dst])
    return res


def _bn_relu(z, g, b):
    m = jnp.mean(z, 0)
    v = jnp.mean((z - m) ** 2, 0)
    return jax.nn.relu((z - m) * lax.rsqrt(v + 1e-5) * g + b)


def _head_body(feat_ref, W1, b1, g1, be1, W2, b2, g2, be2, W3, b3, g3, be3,
               Wout, bout, out_ref):
    z = jnp.dot(feat_ref[...], W1[...], preferred_element_type=jnp.float32) + b1[...]
    o = _bn_relu(z, g1[...], be1[...])
    z = jnp.dot(o, W2[...], preferred_element_type=jnp.float32) + b2[...]
    o = _bn_relu(z, g2[...], be2[...])
    z = jnp.dot(o, W3[...], preferred_element_type=jnp.float32) + b3[...]
    o = _bn_relu(z, g3[...], be3[...])
    z = jnp.dot(o, Wout[...], preferred_element_type=jnp.float32) + bout[...]
    out_ref[...] = jax.nn.sigmoid(z)


def _head(feat, W1, b1, g1, be1, W2, b2, g2, be2, W3, b3, g3, be3, Wout, bout):
    B = feat.shape[0]
    return pl.pallas_call(
        _head_body,
        out_shape=jax.ShapeDtypeStruct((B, 1), jnp.float32),
    )(feat, W1, b1, g1, be1, W2, b2, g2, be2, W3, b3, g3, be3, Wout, bout)


def _hgcn(h, srcs, dsts, Wp, bp, Ws, Wn, bs):
    ms = []
    for i, (name, st, dt) in enumerate(_RELS):
        ms.append(jax.nn.relu(h[st] @ Wp[i] + bp[i]))
    neighs = _segment_max_layer(ms, srcs, dsts)
    out = {nt: jnp.zeros((_NNODES[nt], H), jnp.float32) for nt in _NNODES}
    for i, (name, st, dt) in enumerate(_RELS):
        out[dt] = out[dt] + jax.nn.relu(
            h[dt] @ Ws[i] + neighs[i] @ Wn[i] + bs[i])
    return out


def kernel(x_dr, x_p, finger_feats, seq_feats, disease_feat, e_d_t_dr, e_d_m_dr,
           e_d_p, e_dr_t_d, e_dr_m_d, e_p_d, e_DDI, e_PPI, W_fing, b_fing,
           W_seq, b_seq, W_dis, b_dis, Wp, bp, Ws, Wn, bs, W1, b1, g1, be1,
           W2, b2, g2, be2, W3, b3, g3, be3, Wout, bout):
    edges = [e_d_t_dr, e_d_m_dr, e_d_p, e_dr_t_d, e_dr_m_d, e_p_d, e_DDI,
             e_PPI]
    srcs, dsts = [], []
    for e in edges:
        ne = e.shape[0 + 1]
        pad = _rup(ne, _C) - ne
        srcs.append(jnp.concatenate(
            [e[0], jnp.zeros((pad,), e.dtype)]).astype(jnp.int32))
        dsts.append(jnp.concatenate(
            [e[1], jnp.full((pad,), -1, e.dtype)]).astype(jnp.int32))

    h_dr_f = jax.nn.relu(finger_feats @ W_fing + b_fing)
    h_p_s = jax.nn.relu(seq_feats @ W_seq + b_seq)
    h_d = jax.nn.relu(disease_feat @ W_dis + b_dis)
    h0 = {"drug": h_dr_f, "protein": h_p_s, "disease": h_d}
    h1 = _hgcn(h0, srcs, dsts, Wp, bp, Ws, Wn, bs)
    h2 = _hgcn(h1, srcs, dsts, Wp, bp, Ws, Wn, bs)
    dr_new = jnp.concatenate([h_dr_f, h1["drug"], h2["drug"]], axis=1)
    p_new = jnp.concatenate([h_p_s, h1["protein"], h2["protein"]], axis=1)
    feat = jnp.concatenate([dr_new[x_dr], p_new[x_p]], axis=1)
    return _head(feat, W1, b1, g1, be1, W2, b2, g2, be2, W3, b3, g3, be3,
                 Wout, bout)
